# baseline (speedup 1.0000x reference)
import numpy as np
import jax
import jax.numpy as jnp
from jax.experimental import pallas as pl
from jax.experimental.pallas import tpu as pltpu

_SPLIT = 4
_NSPLIT = 4
_F = _SPLIT * _NSPLIT

_M32 = np.uint32(0xFFFFFFFF)


def _np_threefry2x32(k0, k1, x0, x1):
    x0 = np.asarray(x0, np.uint32).copy()
    x1 = np.asarray(x1, np.uint32).copy()
    ks0, ks1 = np.uint32(k0), np.uint32(k1)
    ks2 = np.uint32(ks0 ^ ks1 ^ np.uint32(0x1BD11BDA))
    rot = ((13, 15, 26, 6), (17, 29, 16, 24))
    inject = ((ks1, ks2), (ks2, ks0), (ks0, ks1), (ks1, ks2), (ks2, ks0))
    x0 = (x0 + ks0) & _M32
    x1 = (x1 + ks1) & _M32
    for i in range(5):
        for r in rot[i % 2]:
            x0 = (x0 + x1) & _M32
            x1 = ((x1 << np.uint32(r)) | (x1 >> np.uint32(32 - r))) & _M32
            x1 = x1 ^ x0
        a, b = inject[i]
        x0 = (x0 + a) & _M32
        x1 = (x1 + b + np.uint32(i + 1)) & _M32
    return x0, x1


def _fold_key(k0, k1, data):
    o0, o1 = _np_threefry2x32(k0, k1, [0], [data])
    return int(o0[0]), int(o1[0])


_KEYS = tuple(_fold_key(0, 42, s) for s in range(_NSPLIT))


def _lane_tables(hwf, w):
    lane = np.arange(hwf, dtype=np.uint32)
    c = lane & np.uint32(_SPLIT - 1)
    s = ((lane >> np.uint32(2)) & np.uint32(_NSPLIT - 1)).astype(np.int64)
    k0 = np.array([k[0] for k in _KEYS], np.uint32)[s]
    k1 = np.array([k[1] for k in _KEYS], np.uint32)[s]
    k2 = k0 ^ k1 ^ np.uint32(0x1BD11BDA)
    hw = lane >> np.uint32(4)
    base = hw * np.uint32(_SPLIT) + c
    rows = [k0, k1, k2, (base + k1) & _M32,
            (k2 + np.uint32(1)) & _M32, (k0 + np.uint32(2)) & _M32,
            (k1 + np.uint32(3)) & _M32, (k2 + np.uint32(4)) & _M32,
            (k0 + np.uint32(5)) & _M32]
    return np.stack(rows)


def _offs_tables(hwf):
    lane = np.arange(hwf)
    c = lane & (_SPLIT - 1)
    ninf = np.float32(-np.inf)
    rows = [np.where(c >= 1, 0.0, ninf), np.where(c >= 2, 0.0, ninf),
            np.where(c >= 3, 0.0, ninf), np.where(c <= 2, 0.0, ninf),
            np.where(c <= 1, 0.0, ninf), np.where(c <= 0, 0.0, ninf)]
    return np.stack(rows).astype(np.float32)


def _sample_body(x_ref, tbl_ref, offs_ref, rowbase_ref, o_ref):
    x = x_ref[...]

    def trow(i):
        return tbl_ref[i:i + 1, :]

    x0 = jnp.broadcast_to(trow(0), x.shape)
    x1 = rowbase_ref[...] + trow(3)
    rot = ((13, 15, 26, 6), (17, 29, 16, 24))
    x0_inj = (1, 2, 0, 1, 2)
    for i in range(5):
        for r in rot[i % 2]:
            x0 = x0 + x1
            x1 = (x1 << jnp.uint32(r)) | (x1 >> jnp.uint32(32 - r))
            x1 = x1 ^ x0
        x0 = x0 + trow(x0_inj[i])
        x1 = x1 + trow(4 + i)
    bits = x0 ^ x1

    tiny = jnp.float32(np.finfo(np.float32).tiny)
    fb = (bits >> jnp.uint32(9)) | jnp.uint32(0x3F800000)
    fl = jax.lax.bitcast_convert_type(fb, jnp.float32) - jnp.float32(1.0)
    u = fl + tiny
    val = x + (-jnp.log(-jnp.log(u)))

    def orow(i):
        return offs_ref[i:i + 1, :]

    def sh(k):
        return jnp.roll(val, k, axis=1)

    ok = (val > sh(1) + orow(0)) & (val > sh(2) + orow(1))
    ok &= (val > sh(3) + orow(2))
    ok &= (val >= sh(-1) + orow(3)) & (val >= sh(-2) + orow(4))
    ok &= (val >= sh(-3) + orow(5))

    o_ref[...] = ok.astype(jnp.float32)


@jax.jit
def kernel(reconstructed_state_logits):
    logits = reconstructed_state_logits
    squeeze = logits.ndim == 3
    if squeeze:
        logits = logits[None, ...]
    B, H, W, F = logits.shape
    assert F == _F

    hwf = H * W * F
    x = logits.reshape(B, hwf)
    b_block = next(n for n in (64, 32, 16, 8, 4, 2, 1) if B % n == 0)
    grid = (B // b_block,)

    tbl = jnp.asarray(_lane_tables(hwf, W))
    offs = jnp.asarray(_offs_tables(hwf))
    rowbase = (jnp.arange(B, dtype=jnp.uint32) * (H * W * _SPLIT)).reshape(
        B, 1)

    out = pl.pallas_call(
        _sample_body,
        grid=grid,
        in_specs=[
            pl.BlockSpec((b_block, hwf), lambda i: (i, 0)),
            pl.BlockSpec((9, hwf), lambda i: (0, 0)),
            pl.BlockSpec((6, hwf), lambda i: (0, 0)),
            pl.BlockSpec((b_block, 1), lambda i: (i, 0)),
        ],
        out_specs=pl.BlockSpec((b_block, hwf), lambda i: (i, 0)),
        out_shape=jax.ShapeDtypeStruct((B, hwf), jnp.float32),
        compiler_params=pltpu.CompilerParams(
            dimension_semantics=("arbitrary",),
            vmem_limit_bytes=100 * 1024 * 1024),
    )(x, tbl, offs, rowbase)

    out = out.reshape(B, H, W, F)
    if squeeze:
        out = out[0]
    return out


# speedup vs baseline: 1.0359x; 1.0359x over previous
import numpy as np
import jax
import jax.numpy as jnp
from jax.experimental import pallas as pl
from jax.experimental.pallas import tpu as pltpu

_SPLIT = 4
_NSPLIT = 4
_F = _SPLIT * _NSPLIT

_M32 = np.uint32(0xFFFFFFFF)


def _np_threefry2x32(k0, k1, x0, x1):
    x0 = np.asarray(x0, np.uint32).copy()
    x1 = np.asarray(x1, np.uint32).copy()
    ks0, ks1 = np.uint32(k0), np.uint32(k1)
    ks2 = np.uint32(ks0 ^ ks1 ^ np.uint32(0x1BD11BDA))
    rot = ((13, 15, 26, 6), (17, 29, 16, 24))
    inject = ((ks1, ks2), (ks2, ks0), (ks0, ks1), (ks1, ks2), (ks2, ks0))
    x0 = (x0 + ks0) & _M32
    x1 = (x1 + ks1) & _M32
    for i in range(5):
        for r in rot[i % 2]:
            x0 = (x0 + x1) & _M32
            x1 = ((x1 << np.uint32(r)) | (x1 >> np.uint32(32 - r))) & _M32
            x1 = x1 ^ x0
        a, b = inject[i]
        x0 = (x0 + a) & _M32
        x1 = (x1 + b + np.uint32(i + 1)) & _M32
    return x0, x1


def _fold_key(k0, k1, data):
    o0, o1 = _np_threefry2x32(k0, k1, [0], [data])
    return int(o0[0]), int(o1[0])


_KEYS = tuple(_fold_key(0, 42, s) for s in range(_NSPLIT))


def _lane_tables(wf):
    lane = np.arange(wf, dtype=np.uint32)
    f = lane & np.uint32(_F - 1)
    c = lane & np.uint32(_SPLIT - 1)
    s = (f >> np.uint32(2)).astype(np.int64)
    k0 = np.array([k[0] for k in _KEYS], np.uint32)[s]
    k1 = np.array([k[1] for k in _KEYS], np.uint32)[s]
    k2 = k0 ^ k1 ^ np.uint32(0x1BD11BDA)
    base = (lane >> np.uint32(4)) * np.uint32(_SPLIT) + c
    rows = [k0, k1, k2, (base + k1) & _M32,
            (k2 + np.uint32(1)) & _M32, (k0 + np.uint32(2)) & _M32,
            (k1 + np.uint32(3)) & _M32, (k2 + np.uint32(4)) & _M32,
            (k0 + np.uint32(5)) & _M32]
    return np.stack(rows)[:, None, :]


def _offs_tables(wf):
    lane = np.arange(wf)
    c = lane & (_SPLIT - 1)
    ninf = np.float32(-np.inf)
    rows = [np.where(c >= 1, 0.0, ninf), np.where(c >= 2, 0.0, ninf),
            np.where(c >= 3, 0.0, ninf), np.where(c <= 2, 0.0, ninf),
            np.where(c <= 1, 0.0, ninf), np.where(c <= 0, 0.0, ninf)]
    return np.stack(rows).astype(np.float32)[:, None, :]


def _sample_body(x_ref, tbl_ref, offs_ref, rowbase_ref, o_ref):
    x = x_ref[...]

    def trow(i):
        return tbl_ref[i:i + 1, 0:1, :]

    x0 = jnp.broadcast_to(trow(0), x.shape)
    x1 = rowbase_ref[...] + trow(3)
    rot = ((13, 15, 26, 6), (17, 29, 16, 24))
    x0_inj = (1, 2, 0, 1, 2)
    for i in range(5):
        for r in rot[i % 2]:
            x0 = x0 + x1
            x1 = (x1 << jnp.uint32(r)) | (x1 >> jnp.uint32(32 - r))
            x1 = x1 ^ x0
        x0 = x0 + trow(x0_inj[i])
        x1 = x1 + trow(4 + i)
    bits = x0 ^ x1

    tiny = jnp.float32(np.finfo(np.float32).tiny)
    fb = (bits >> jnp.uint32(9)) | jnp.uint32(0x3F800000)
    fl = jax.lax.bitcast_convert_type(fb, jnp.float32) - jnp.float32(1.0)
    u = fl + tiny
    val = x + (-jnp.log(-jnp.log(u)))

    def orow(i):
        return offs_ref[i:i + 1, 0:1, :]

    def sh(k):
        return jnp.roll(val, k, axis=2)

    ok = (val > sh(1) + orow(0)) & (val > sh(2) + orow(1))
    ok &= (val > sh(3) + orow(2))
    ok &= (val >= sh(-1) + orow(3)) & (val >= sh(-2) + orow(4))
    ok &= (val >= sh(-3) + orow(5))

    o_ref[...] = ok.astype(jnp.float32)


@jax.jit
def kernel(reconstructed_state_logits):
    logits = reconstructed_state_logits
    squeeze = logits.ndim == 3
    if squeeze:
        logits = logits[None, ...]
    B, H, W, F = logits.shape
    assert F == _F

    wf = W * F
    x = logits.reshape(B, H, wf)
    b_block = next(n for n in (64, 32, 16, 8, 4, 2, 1) if B % n == 0)
    grid = (B // b_block,)

    tbl = jnp.asarray(_lane_tables(wf))
    offs = jnp.asarray(_offs_tables(wf))
    rowbase = (jnp.arange(B * H, dtype=jnp.uint32) * (W * _SPLIT)).reshape(
        B, H, 1)

    out = pl.pallas_call(
        _sample_body,
        grid=grid,
        in_specs=[
            pl.BlockSpec((b_block, H, wf), lambda i: (i, 0, 0)),
            pl.BlockSpec((9, 1, wf), lambda i: (0, 0, 0)),
            pl.BlockSpec((6, 1, wf), lambda i: (0, 0, 0)),
            pl.BlockSpec((b_block, H, 1), lambda i: (i, 0, 0)),
        ],
        out_specs=pl.BlockSpec((b_block, H, wf), lambda i: (i, 0, 0)),
        out_shape=jax.ShapeDtypeStruct((B, H, wf), jnp.float32),
        compiler_params=pltpu.CompilerParams(
            dimension_semantics=("arbitrary",),
            vmem_limit_bytes=100 * 1024 * 1024),
    )(x, tbl, offs, rowbase)

    out = out.reshape(B, H, W, F)
    if squeeze:
        out = out[0]
    return out
